# baseline (speedup 1.0000x reference)
import functools

import jax
import jax.numpy as jnp
from jax import lax
from jax.experimental import pallas as pl
from jax.experimental.pallas import tpu as pltpu
from jax.experimental.pallas import tpu_sc as plsc

NB_NEIGHBORS = 16
NB_FEATURES = 1024
BATCH = 1024
GATHER_COLS = NB_FEATURES * NB_NEIGHBORS
OUT_COLS = 1 + GATHER_COLS
COLS_PAD = 16512

NUM_WORKERS = 32
ROWS_PER_W = BATCH // NUM_WORKERS
COL_BLOCK = 1024
N_BLOCKS = GATHER_COLS // COL_BLOCK
CHUNKS_PER_BLOCK = COL_BLOCK // 16


def _topk_body(coords_ref, nbr_ref):
    X = coords_ref[...]
    xtx = lax.dot_general(
        X, X, (((0,), (0,)), ((), ())), preferred_element_type=jnp.float32
    )
    xx = jnp.sum(jnp.square(X), axis=0)
    d = xtx * -2.0
    d = d + xx[None, :]
    d = d + xx[:, None]
    d = jnp.maximum(d, 0.0)
    d = jnp.sqrt(d)

    lane = lax.broadcasted_iota(
        jnp.int32, (NB_FEATURES, NB_FEATURES), 1).astype(jnp.float32)
    klane = lax.broadcasted_iota(jnp.int32, (NB_FEATURES, NB_NEIGHBORS), 1)
    nbr = jnp.zeros((NB_FEATURES, NB_NEIGHBORS), dtype=jnp.float32)
    work = d
    big = jnp.float32(2**30)
    for k in range(NB_NEIGHBORS):
        m = jnp.min(work, axis=1, keepdims=True)
        idx = jnp.min(jnp.where(work == m, lane, big), axis=1, keepdims=True)
        nbr = jnp.where(klane == k, idx, nbr)
        work = jnp.where(lane == idx, jnp.float32(jnp.inf), work)
    nbr_ref[...] = nbr.astype(jnp.int32)


def _sc_gather(in_hbm, cols_hbm, out_hbm, cols_v, in_v, buf_a, buf_b,
               colbuf, sem_a, sem_b):
    wid = lax.axis_index("s") * 2 + lax.axis_index("c")
    row0 = wid * ROWS_PER_W
    pltpu.sync_copy(cols_hbm, cols_v)
    pltpu.sync_copy(in_hbm.at[pl.ds(row0 * NB_FEATURES, ROWS_PER_W * NB_FEATURES)],
                    in_v)

    zeros16 = jnp.zeros((16,), jnp.int32)

    def fill(cb, buf):
        @plsc.parallel_loop(0, CHUNKS_PER_BLOCK, unroll=4)
        def chunk_body(c):
            idx = cols_v[pl.ds(cb * COL_BLOCK + c * 16, 16)]
            for b in range(ROWS_PER_W):
                vals = plsc.load_gather(in_v, [idx + jnp.int32(b * NB_FEATURES)])
                buf[b, pl.ds(c * 16, 16)] = vals

    def out_block(cb):
        return out_hbm.at[pl.ds(row0, ROWS_PER_W),
                          pl.ds(cb * COL_BLOCK, COL_BLOCK)]

    def pair_body(p, carry):
        for phase, (buf, sem) in enumerate(((buf_a, sem_a), (buf_b, sem_b))):
            cb = p * 2 + phase

            @pl.when(p > 0)
            def _wait():
                pltpu.make_async_copy(buf, out_block(cb - 2), sem).wait()

            fill(cb, buf)
            pltpu.async_copy(buf, out_block(cb), sem)
        return carry

    lax.fori_loop(0, N_BLOCKS // 2, pair_body, 0)

    cstar = plsc.load_gather(cols_v, [jnp.full((16,), GATHER_COLS, jnp.int32)])
    for h in range(2):
        rows = lax.iota(jnp.int32, 16) + jnp.int32(h * 16)
        vals = plsc.load_gather(in_v, [cstar + rows * NB_FEATURES])
        plsc.store_scatter(colbuf, [rows, zeros16], vals)
    pltpu.make_async_copy(buf_a, out_block(N_BLOCKS - 2), sem_a).wait()
    pltpu.make_async_copy(buf_b, out_block(N_BLOCKS - 1), sem_b).wait()
    pltpu.sync_copy(colbuf, out_hbm.at[pl.ds(row0, ROWS_PER_W),
                                       pl.ds(GATHER_COLS, 1)])


@jax.jit
def kernel(inputs, coordinates):
    nbr = pl.pallas_call(
        _topk_body,
        out_shape=jax.ShapeDtypeStruct((NB_FEATURES, NB_NEIGHBORS), jnp.int32),
    )(coordinates)

    cols = jnp.concatenate([
        jnp.zeros((1,), dtype=jnp.int32),
        nbr.reshape(-1),
        jnp.zeros((COLS_PAD - OUT_COLS,), dtype=jnp.int32),
    ])
    in_flat = inputs.reshape(-1)

    sc_call = functools.partial(
        pl.kernel,
        mesh=plsc.VectorSubcoreMesh(core_axis_name="c", subcore_axis_name="s"),
        compiler_params=pltpu.CompilerParams(needs_layout_passes=False),
        out_type=jax.ShapeDtypeStruct((BATCH, OUT_COLS), jnp.float32),
        scratch_types=[
            pltpu.VMEM((COLS_PAD,), jnp.int32),
            pltpu.VMEM((ROWS_PER_W * NB_FEATURES,), jnp.float32),
            pltpu.VMEM((ROWS_PER_W, COL_BLOCK), jnp.float32),
            pltpu.VMEM((ROWS_PER_W, COL_BLOCK), jnp.float32),
            pltpu.VMEM((ROWS_PER_W, 1), jnp.float32),
            pltpu.SemaphoreType.DMA,
            pltpu.SemaphoreType.DMA,
        ],
    )(_sc_gather)
    out = sc_call(in_flat, cols)
    return out[:, None, :, None]


# speedup vs baseline: 1.0940x; 1.0940x over previous
import functools

import jax
import jax.numpy as jnp
from jax import lax
from jax.experimental import pallas as pl
from jax.experimental.pallas import tpu as pltpu
from jax.experimental.pallas import tpu_sc as plsc

NB_NEIGHBORS = 16
NB_FEATURES = 1024
BATCH = 1024
GATHER_COLS = NB_FEATURES * NB_NEIGHBORS
OUT_COLS = 1 + GATHER_COLS
COLS_PAD = 16512

NUM_WORKERS = 32
ROWS_PER_W = BATCH // NUM_WORKERS
COL_BLOCK = 512
N_BLOCKS = GATHER_COLS // COL_BLOCK
CHUNKS_PER_BLOCK = COL_BLOCK // 16


def _topk_body(coords_ref, nbr_ref):
    X = coords_ref[...]
    xtx = lax.dot_general(
        X, X, (((0,), (0,)), ((), ())), preferred_element_type=jnp.float32
    )
    xx = jnp.sum(jnp.square(X), axis=0)
    d = xtx * -2.0
    d = d + xx[None, :]
    d = d + xx[:, None]
    d = jnp.maximum(d, 0.0)
    d = jnp.sqrt(d)

    lane = lax.broadcasted_iota(
        jnp.int32, (NB_FEATURES, NB_FEATURES), 1).astype(jnp.float32)
    klane = lax.broadcasted_iota(jnp.int32, (NB_FEATURES, NB_NEIGHBORS), 1)
    nbr = jnp.zeros((NB_FEATURES, NB_NEIGHBORS), dtype=jnp.float32)
    work = d
    big = jnp.float32(2**30)
    for k in range(NB_NEIGHBORS):
        m = jnp.min(work, axis=1, keepdims=True)
        idx = jnp.min(jnp.where(work == m, lane, big), axis=1, keepdims=True)
        nbr = jnp.where(klane == k, idx, nbr)
        work = jnp.where(lane == idx, jnp.float32(jnp.inf), work)
    nbr_ref[...] = nbr.astype(jnp.int32)


def _sc_gather(in_hbm, cols_hbm, out_hbm, cols_v, in_v, buf_a, buf_b,
               colbuf, sem_a, sem_b):
    wid = lax.axis_index("s") * 2 + lax.axis_index("c")
    row0 = wid * ROWS_PER_W
    pltpu.sync_copy(cols_hbm, cols_v)
    pltpu.sync_copy(in_hbm.at[pl.ds(row0 * NB_FEATURES, ROWS_PER_W * NB_FEATURES)],
                    in_v)

    zeros16 = jnp.zeros((16,), jnp.int32)

    def fill(cb, buf):
        @plsc.parallel_loop(0, CHUNKS_PER_BLOCK, unroll=2)
        def chunk_body(c):
            idx = cols_v[pl.ds(cb * COL_BLOCK + c * 16, 16)]
            for b in range(ROWS_PER_W):
                vals = plsc.load_gather(in_v, [idx + jnp.int32(b * NB_FEATURES)])
                buf[b, pl.ds(c * 16, 16)] = vals

    def out_block(cb):
        return out_hbm.at[pl.ds(row0, ROWS_PER_W),
                          pl.ds(cb * COL_BLOCK, COL_BLOCK)]

    def pair_body(p, carry):
        for phase, (buf, sem) in enumerate(((buf_a, sem_a), (buf_b, sem_b))):
            cb = p * 2 + phase

            @pl.when(p > 0)
            def _wait():
                pltpu.make_async_copy(buf, out_block(cb - 2), sem).wait()

            fill(cb, buf)
            pltpu.async_copy(buf, out_block(cb), sem)
        return carry

    lax.fori_loop(0, N_BLOCKS // 2, pair_body, 0)

    cstar = plsc.load_gather(cols_v, [jnp.full((16,), GATHER_COLS, jnp.int32)])
    for h in range(2):
        rows = lax.iota(jnp.int32, 16) + jnp.int32(h * 16)
        vals = plsc.load_gather(in_v, [cstar + rows * NB_FEATURES])
        plsc.store_scatter(colbuf, [rows, zeros16], vals)
    pltpu.make_async_copy(buf_a, out_block(N_BLOCKS - 2), sem_a).wait()
    pltpu.make_async_copy(buf_b, out_block(N_BLOCKS - 1), sem_b).wait()
    pltpu.sync_copy(colbuf, out_hbm.at[pl.ds(row0, ROWS_PER_W),
                                       pl.ds(GATHER_COLS, 1)])


@jax.jit
def kernel(inputs, coordinates):
    nbr = pl.pallas_call(
        _topk_body,
        out_shape=jax.ShapeDtypeStruct((NB_FEATURES, NB_NEIGHBORS), jnp.int32),
    )(coordinates)

    cols = jnp.concatenate([
        jnp.zeros((1,), dtype=jnp.int32),
        nbr.reshape(-1),
        jnp.zeros((COLS_PAD - OUT_COLS,), dtype=jnp.int32),
    ])
    in_flat = inputs.reshape(-1)

    sc_call = functools.partial(
        pl.kernel,
        mesh=plsc.VectorSubcoreMesh(core_axis_name="c", subcore_axis_name="s"),
        compiler_params=pltpu.CompilerParams(needs_layout_passes=False),
        out_type=jax.ShapeDtypeStruct((BATCH, OUT_COLS), jnp.float32),
        scratch_types=[
            pltpu.VMEM((COLS_PAD,), jnp.int32),
            pltpu.VMEM((ROWS_PER_W * NB_FEATURES,), jnp.float32),
            pltpu.VMEM((ROWS_PER_W, COL_BLOCK), jnp.float32),
            pltpu.VMEM((ROWS_PER_W, COL_BLOCK), jnp.float32),
            pltpu.VMEM((ROWS_PER_W, 1), jnp.float32),
            pltpu.SemaphoreType.DMA,
            pltpu.SemaphoreType.DMA,
        ],
    )(_sc_gather)
    out = sc_call(in_flat, cols)
    return out[:, None, :, None]
